# baseline (speedup 1.0000x reference)
import functools

import jax
import jax.numpy as jnp
from jax import lax
from jax.experimental import pallas as pl
from jax.experimental.pallas import tpu as pltpu
from jax.experimental.pallas import tpu_sc as plsc

NC = 2
NS = 16
CHUNK = 64


def _mm_split_body(n, dh, x_ref, w_ref, o_ref):
    r = jnp.dot(x_ref[...], w_ref[...], preferred_element_type=jnp.float32)
    o_ref[0, :n] = r[:, :dh]
    o_ref[1, :n] = r[:, dh:]


def _matmul_split(x, w, n_pad):
    n = x.shape[0]
    dout = w.shape[1]
    dh = dout // 2
    return pl.pallas_call(
        functools.partial(_mm_split_body, n, dh),
        out_shape=jax.ShapeDtypeStruct((2, n_pad, dh), jnp.float32),
    )(x, w)


def _relu_mm_body(n, dh, p_ref, w_ref, o_ref):
    h = jnp.maximum(p_ref[:n], 0.0)
    r = jnp.dot(h, w_ref[...], preferred_element_type=jnp.float32)
    o_ref[0, :n] = r[:, :dh]
    o_ref[1, :n] = r[:, dh:]


def _relu_mm(parts, w, n, n_pad):
    dout = w.shape[1]
    dh = dout // 2
    return pl.pallas_call(
        functools.partial(_relu_mm_body, n, dh),
        out_shape=jax.ShapeDtypeStruct((2, n_pad, dh), jnp.float32),
    )(parts, w)


def _make_aggregate(n, n_pad, dh, chunks):
    mesh = plsc.VectorSubcoreMesh(core_axis_name="c", subcore_axis_name="s")
    rows_per_sub = n_pad // NS

    @functools.partial(
        pl.kernel,
        mesh=mesh,
        compiler_params=pltpu.CompilerParams(use_tc_tiling_on_sc=False),
        out_type=jax.ShapeDtypeStruct((n_pad, 2 * dh), jnp.float32),
        scratch_types=[
            pltpu.VMEM((chunks // 5, CHUNK), jnp.int32),
            pltpu.VMEM((chunks // 5, CHUNK), jnp.int32),
            pltpu.VMEM((CHUNK, dh), jnp.float32),
            pltpu.VMEM((CHUNK, dh), jnp.float32),
            pltpu.VMEM((CHUNK, dh), jnp.float32),
            pltpu.VMEM((CHUNK, dh), jnp.float32),
            pltpu.VMEM_SHARED((2 * n_pad, dh), jnp.float32),
            pltpu.SemaphoreType.DMA,
            pltpu.SemaphoreType.DMA,
            pltpu.SemaphoreType.DMA,
            pltpu.SemaphoreType.DMA,
            pltpu.SemaphoreType.DMA,
            pltpu.SemaphoreType.DMA,
            pltpu.SemaphoreType.DMA,
            pltpu.SemaphoreType.DMA,
        ],
    )
    def aggregate(h_hbm, src_hbm, dst_hbm, init_hbm, out_hbm,
                  src_v, dst_v, r0, r1, r2, r3, sp,
                  g0, g1, g2, g3, s0, s1, s2, s3):
        c = lax.axis_index("c")
        s = lax.axis_index("s")
        row0 = s * rows_per_sub
        pltpu.sync_copy(h_hbm.at[c, pl.ds(row0, rows_per_sub)],
                        sp.at[pl.ds(row0, rows_per_sub)])
        pltpu.sync_copy(init_hbm.at[c, pl.ds(row0, rows_per_sub)],
                        sp.at[pl.ds(n_pad + row0, rows_per_sub)])

        plsc.subcore_barrier()
        cq = chunks // 5
        rows = (r0, r1, r2, r3)
        gsem = (g0, g1, g2, g3)
        ssem = (s0, s1, s2, s3)

        def body(i, carry):
            for k in range(4):
                j = 4 * i + k
                pltpu.make_async_copy(sp.at[src_v.at[j]], rows[k],
                                      gsem[k]).wait()
                pltpu.async_copy(rows[k], sp.at[dst_v.at[j]], ssem[k],
                                 add=True)
                pltpu.make_async_copy(rows[k], sp.at[dst_v.at[j]],
                                      ssem[k]).wait()
                pltpu.async_copy(sp.at[src_v.at[jnp.minimum(j + 4, cq - 1)]],
                                 rows[k], gsem[k])
            return carry

        for q in range(5):
            pltpu.sync_copy(src_hbm.at[s, pl.ds(q * cq, cq)], src_v)
            pltpu.sync_copy(dst_hbm.at[s, pl.ds(q * cq, cq)], dst_v)
            for k in range(4):
                pltpu.async_copy(sp.at[src_v.at[k]], rows[k], gsem[k])
            lax.fori_loop(0, cq // 4, body, 0)
            for k in range(4):
                pltpu.make_async_copy(sp.at[src_v.at[cq - 1]], rows[k],
                                      gsem[k]).wait()
        plsc.subcore_barrier()
        pltpu.sync_copy(sp.at[pl.ds(n_pad + row0, rows_per_sub)],
                        out_hbm.at[pl.ds(row0, rows_per_sub),
                                   pl.ds(c * dh, dh)])

    return aggregate


def kernel(x, edge_index, W1, b1, W2, b2):
    n, d = x.shape
    dh = d // 2
    e = edge_index.shape[1]

    chunks = -(-(-(-e // (NS * CHUNK))) // 32) * 32
    e_pad = NS * chunks * CHUNK
    n_pad = -(-(n + 1) // (NS * 8)) * (NS * 8)
    src = jnp.concatenate(
        [edge_index[0], jnp.zeros((e_pad - e,), jnp.int32)]).reshape(
            NS, chunks, CHUNK)
    dst = jnp.concatenate(
        [edge_index[1] + n_pad, jnp.full((e_pad - e,), n_pad + n,
                                         jnp.int32)]).reshape(
            NS, chunks, CHUNK)
    init1 = jnp.broadcast_to(b1.reshape(2, 1, dh), (2, n_pad, dh))
    init2 = jnp.broadcast_to(b2.reshape(2, 1, dh), (2, n_pad, dh))

    aggregate = _make_aggregate(n, n_pad, dh, chunks)

    h1 = _matmul_split(x, W1, n_pad)
    p1 = aggregate(h1, src, dst, init1)
    h2 = _relu_mm(p1, W2, n, n_pad)
    p2 = aggregate(h2, src, dst, init2)
    return p2[:n]


# speedup vs baseline: 1.0081x; 1.0081x over previous
import functools

import jax
import jax.numpy as jnp
from jax import lax
from jax.experimental import pallas as pl
from jax.experimental.pallas import tpu as pltpu
from jax.experimental.pallas import tpu_sc as plsc

NC = 2
NS = 16
CHUNK = 64


def _mm_split_body(n, dh, x_ref, w_ref, o_ref):
    r = jnp.dot(x_ref[...], w_ref[...], preferred_element_type=jnp.float32)
    o_ref[0, :n] = r[:, :dh]
    o_ref[1, :n] = r[:, dh:]


def _matmul_split(x, w, n_pad):
    n = x.shape[0]
    dout = w.shape[1]
    dh = dout // 2
    return pl.pallas_call(
        functools.partial(_mm_split_body, n, dh),
        out_shape=jax.ShapeDtypeStruct((2, n_pad, dh), jnp.float32),
    )(x, w)


def _relu_mm_body(n, dh, p_ref, w_ref, o_ref):
    h = jnp.maximum(p_ref[:n], 0.0)
    r = jnp.dot(h, w_ref[...], preferred_element_type=jnp.float32)
    o_ref[0, :n] = r[:, :dh]
    o_ref[1, :n] = r[:, dh:]


def _relu_mm(parts, w, n, n_pad):
    dout = w.shape[1]
    dh = dout // 2
    return pl.pallas_call(
        functools.partial(_relu_mm_body, n, dh),
        out_shape=jax.ShapeDtypeStruct((2, n_pad, dh), jnp.float32),
    )(parts, w)


def _make_aggregate(n, n_pad, dh, chunks):
    mesh = plsc.VectorSubcoreMesh(core_axis_name="c", subcore_axis_name="s")
    rows_per_sub = n_pad // NS

    @functools.partial(
        pl.kernel,
        mesh=mesh,
        compiler_params=pltpu.CompilerParams(use_tc_tiling_on_sc=False),
        out_type=jax.ShapeDtypeStruct((n_pad, 2 * dh), jnp.float32),
        scratch_types=[
            pltpu.VMEM((chunks // 4, CHUNK), jnp.int32),
            pltpu.VMEM((chunks // 4, CHUNK), jnp.int32),
            pltpu.VMEM((CHUNK, dh), jnp.float32),
            pltpu.VMEM((CHUNK, dh), jnp.float32),
            pltpu.VMEM_SHARED((2 * n_pad, dh), jnp.float32),
            pltpu.SemaphoreType.DMA,
            pltpu.SemaphoreType.DMA,
            pltpu.SemaphoreType.DMA,
            pltpu.SemaphoreType.DMA,
        ],
    )
    def aggregate(h_hbm, src_hbm, dst_hbm, init_hbm, out_hbm,
                  src_v, dst_v, rows_a, rows_b, sp, g_a, g_b, s_a, s_b):
        c = lax.axis_index("c")
        s = lax.axis_index("s")
        row0 = s * rows_per_sub
        pltpu.sync_copy(h_hbm.at[c, pl.ds(row0, rows_per_sub)],
                        sp.at[pl.ds(row0, rows_per_sub)])
        pltpu.sync_copy(init_hbm.at[c, pl.ds(row0, rows_per_sub)],
                        sp.at[pl.ds(n_pad + row0, rows_per_sub)])

        plsc.subcore_barrier()
        cq = chunks // 4

        def body(i, carry):
            j0 = 2 * i
            j1 = j0 + 1
            pltpu.make_async_copy(sp.at[src_v.at[j0]], rows_a, g_a).wait()
            pltpu.async_copy(rows_a, sp.at[dst_v.at[j0]], s_a, add=True)
            pltpu.make_async_copy(rows_a, sp.at[dst_v.at[j0]], s_a).wait()
            pltpu.async_copy(sp.at[src_v.at[jnp.minimum(j0 + 2, cq - 1)]],
                             rows_a, g_a)
            pltpu.make_async_copy(sp.at[src_v.at[j1]], rows_b, g_b).wait()
            pltpu.async_copy(rows_b, sp.at[dst_v.at[j1]], s_b, add=True)
            pltpu.make_async_copy(rows_b, sp.at[dst_v.at[j1]], s_b).wait()
            pltpu.async_copy(sp.at[src_v.at[jnp.minimum(j1 + 2, cq - 1)]],
                             rows_b, g_b)
            return carry

        for q in range(4):
            pltpu.sync_copy(src_hbm.at[s, pl.ds(q * cq, cq)], src_v)
            pltpu.sync_copy(dst_hbm.at[s, pl.ds(q * cq, cq)], dst_v)
            pltpu.async_copy(sp.at[src_v.at[0]], rows_a, g_a)
            pltpu.async_copy(sp.at[src_v.at[1]], rows_b, g_b)
            lax.fori_loop(0, cq // 2, body, 0)
            pltpu.make_async_copy(sp.at[src_v.at[cq - 1]], rows_a, g_a).wait()
            pltpu.make_async_copy(sp.at[src_v.at[cq - 1]], rows_b, g_b).wait()
        plsc.subcore_barrier()
        pltpu.sync_copy(sp.at[pl.ds(n_pad + row0, rows_per_sub)],
                        out_hbm.at[pl.ds(row0, rows_per_sub),
                                   pl.ds(c * dh, dh)])

    return aggregate


def kernel(x, edge_index, W1, b1, W2, b2):
    n, d = x.shape
    dh = d // 2
    e = edge_index.shape[1]

    chunks = -(-(-(-e // (NS * CHUNK))) // 32) * 32
    e_pad = NS * chunks * CHUNK
    n_pad = -(-(n + 1) // (NS * 8)) * (NS * 8)
    src = jnp.concatenate(
        [edge_index[0], jnp.zeros((e_pad - e,), jnp.int32)]).reshape(
            NS, chunks, CHUNK)
    dst = jnp.concatenate(
        [edge_index[1] + n_pad, jnp.full((e_pad - e,), n_pad + n,
                                         jnp.int32)]).reshape(
            NS, chunks, CHUNK)
    init1 = jnp.broadcast_to(b1.reshape(2, 1, dh), (2, n_pad, dh))
    init2 = jnp.broadcast_to(b2.reshape(2, 1, dh), (2, n_pad, dh))

    aggregate = _make_aggregate(n, n_pad, dh, chunks)

    h1 = _matmul_split(x, W1, n_pad)
    p1 = aggregate(h1, src, dst, init1)
    h2 = _relu_mm(p1, W2, n, n_pad)
    p2 = aggregate(h2, src, dst, init2)
    return p2[:n]
